# baseline (speedup 1.0000x reference)
import jax
import jax.numpy as jnp
from jax import lax
from jax.experimental import pallas as pl
from jax.experimental.pallas import tpu as pltpu
from jax.experimental.pallas import tpu_sc as plsc

NUM_HEADS = 16
SEQ = 2048
TWIDTH = 3968
NCHUNK = TWIDTH // 16


def _body(bias_hbm, delta_hbm, out_hbm, bias_v, delta_v,
          t0, t1, t2, t3, sem0, sem1, sem2, sem3):
    nc = 2
    wid = lax.axis_index("s") * nc + lax.axis_index("c")
    h = wid // 2
    half = wid % 2

    pltpu.sync_copy(bias_hbm.at[pl.ds(h * 256, 256)], bias_v)
    pltpu.sync_copy(delta_hbm, delta_v)
    delta = delta_v[...]

    lane = lax.iota(jnp.int32, 16)
    bufs = (t0, t1, t2, t3)
    sems = (sem0, sem1, sem2, sem3)
    nbuf = len(bufs)

    def drain16(t_q, sem):
        def drain(u, carry):
            pltpu.make_async_copy(
                t_q.at[:, pl.ds(0, SEQ)], out_hbm.at[h, pl.ds(0, 8)], sem).wait()
            return carry
        lax.fori_loop(0, 16, drain, None)

    for p in range(8):
        t_q, sem = bufs[p % nbuf], sems[p % nbuf]
        if p >= nbuf:
            drain16(t_q, sem)

        q = half * 8 + p
        base = 1920 + 8 * q

        for r in range(8):
            def build_chunk(c, carry, r=r, base=base, t_q=t_q):
                a = c * 16 + lane
                idx = jnp.clip(base + r - a + delta, -127, 127) + 127
                t_q[r, pl.ds(c * 16, 16)] = plsc.load_gather(bias_v, [idx])
                return carry
            lax.fori_loop(0, NCHUNK, build_chunk, None)

        def fire(u, carry, q=q, t_q=t_q, sem=sem):
            row0 = 8 * (q + 16 * u)
            s = 128 * (15 - u)
            pltpu.async_copy(
                t_q.at[:, pl.ds(s, SEQ)], out_hbm.at[h, pl.ds(row0, 8)], sem)
            return carry
        lax.fori_loop(0, 16, fire, None)

    for b in range(nbuf):
        drain16(bufs[b], sems[b])


def kernel(bias, qlen, klen):
    bias_flat = jnp.pad(bias, ((0, 0), (0, 1))).reshape(-1)
    delta = jnp.full((16,), qlen - klen, dtype=jnp.int32)

    run = pl.kernel(
        _body,
        out_type=jax.ShapeDtypeStruct((NUM_HEADS, SEQ, SEQ), jnp.float32),
        mesh=plsc.VectorSubcoreMesh(core_axis_name="c", subcore_axis_name="s"),
        compiler_params=pltpu.CompilerParams(needs_layout_passes=False),
        scratch_types=[
            pltpu.VMEM((256,), jnp.float32),
            pltpu.VMEM((16,), jnp.int32),
            pltpu.VMEM((8, TWIDTH), jnp.float32),
            pltpu.VMEM((8, TWIDTH), jnp.float32),
            pltpu.VMEM((8, TWIDTH), jnp.float32),
            pltpu.VMEM((8, TWIDTH), jnp.float32),
            pltpu.SemaphoreType.DMA,
            pltpu.SemaphoreType.DMA,
            pltpu.SemaphoreType.DMA,
            pltpu.SemaphoreType.DMA,
        ],
    )
    return run(bias_flat, delta)


# speedup vs baseline: 1.0116x; 1.0116x over previous
import jax
import jax.numpy as jnp
from jax import lax
from jax.experimental import pallas as pl
from jax.experimental.pallas import tpu as pltpu
from jax.experimental.pallas import tpu_sc as plsc

NUM_HEADS = 16
SEQ = 2048
TWIDTH = 3968
NCHUNK = TWIDTH // 16


def _body(bias_hbm, delta_hbm, out_hbm, bias_v, delta_v, t0, t1, sem0, sem1):
    nc = 2
    wid = lax.axis_index("s") * nc + lax.axis_index("c")
    h = wid // 2
    half = wid % 2

    pltpu.sync_copy(bias_hbm.at[pl.ds(h * 256, 256)], bias_v)
    pltpu.sync_copy(delta_hbm, delta_v)
    delta = delta_v[...]

    lane = lax.iota(jnp.int32, 16)
    bufs = (t0, t1)
    sems = (sem0, sem1)
    nbuf = len(bufs)

    def drain16(t_q, sem):
        def drain(u, carry):
            pltpu.make_async_copy(
                t_q.at[:, pl.ds(0, SEQ)], out_hbm.at[h, pl.ds(0, 8)], sem).wait()
            return carry
        lax.fori_loop(0, 16, drain, None)

    for p in range(8):
        t_q, sem = bufs[p % nbuf], sems[p % nbuf]
        if p >= nbuf:
            drain16(t_q, sem)

        q = half * 8 + p
        base = 1920 + 8 * q

        for r in range(8):
            def build_chunk(c, carry, r=r, base=base, t_q=t_q):
                a = c * 16 + lane
                idx = jnp.clip(base + r - a + delta, -127, 127) + 127
                t_q[r, pl.ds(c * 16, 16)] = plsc.load_gather(bias_v, [idx])
                return carry
            lax.fori_loop(0, NCHUNK, build_chunk, None)

        def fire(u, carry, q=q, t_q=t_q, sem=sem):
            row0 = 8 * (q + 16 * u)
            s = 128 * (15 - u)
            pltpu.async_copy(
                t_q.at[:, pl.ds(s, SEQ)], out_hbm.at[h, pl.ds(row0, 8)], sem)
            return carry
        lax.fori_loop(0, 16, fire, None)

    for b in range(nbuf):
        drain16(bufs[b], sems[b])


def kernel(bias, qlen, klen):
    bias_flat = jnp.pad(bias, ((0, 0), (0, 1))).reshape(-1)
    delta = jnp.full((16,), qlen - klen, dtype=jnp.int32)

    run = pl.kernel(
        _body,
        out_type=jax.ShapeDtypeStruct((NUM_HEADS, SEQ, SEQ), jnp.float32),
        mesh=plsc.VectorSubcoreMesh(core_axis_name="c", subcore_axis_name="s"),
        compiler_params=pltpu.CompilerParams(needs_layout_passes=False),
        scratch_types=[
            pltpu.VMEM((256,), jnp.float32),
            pltpu.VMEM((16,), jnp.int32),
            pltpu.VMEM((8, TWIDTH), jnp.float32),
            pltpu.VMEM((8, TWIDTH), jnp.float32),
            pltpu.SemaphoreType.DMA,
            pltpu.SemaphoreType.DMA,
        ],
    )
    return run(bias_flat, delta)
